# Initial kernel scaffold; baseline (speedup 1.0000x reference)
import jax
import jax.numpy as jnp
from jax.experimental import pallas as pl


def kernel(features_omics1, features_omics2, adj_spatial_omics1, adj_feature_omics1, adj_spatial_omics2, adj_feature_omics2, Ws1, Wn1, Wsd1, Wnd1, Ws2, Wn2, Wsd2, Wnd2, w1, u1, w2, u2, wc, uc):
    raise NotImplementedError("write your pallas kernel here")


# speedup vs baseline: 1.4436x; 1.4436x over previous
import jax
import jax.numpy as jnp
from jax.experimental import pallas as pl
from jax.experimental.pallas import tpu as pltpu


_PAR = pltpu.CompilerParams(dimension_semantics=("parallel",))


def _big_relu_body(a_ref, h_ref, s_ref, o_ref):
    acc = jnp.dot(a_ref[...].astype(jnp.bfloat16), h_ref[...],
                  preferred_element_type=jnp.float32)
    o_ref[...] = jnp.maximum(acc + s_ref[...], 0.0)


def _sage_ref_body(a_ref, f_ref, s_ref, wn_ref, o_ref):
    dot = lambda x, y: jnp.dot(x, y, preferred_element_type=jnp.float32)
    neigh = dot(a_ref[...].astype(jnp.bfloat16), f_ref[...])
    acc = dot(neigh.astype(jnp.bfloat16), wn_ref[...])
    o_ref[...] = jnp.maximum(acc + s_ref[...], 0.0)


def _sage_ref(adj, feat_bf16, s, wn_bf16, block_rows=200):
    n, k = adj.shape
    d = feat_bf16.shape[1]
    o = wn_bf16.shape[1]
    r = block_rows if n % block_rows == 0 else n
    return pl.pallas_call(
        _sage_ref_body, grid=(n // r,),
        in_specs=[pl.BlockSpec((r, k), lambda i: (i, 0)),
                  pl.BlockSpec((k, d), lambda i: (0, 0)),
                  pl.BlockSpec((r, o), lambda i: (i, 0)),
                  pl.BlockSpec((d, o), lambda i: (0, 0))],
        out_specs=pl.BlockSpec((r, o), lambda i: (i, 0)),
        out_shape=jax.ShapeDtypeStruct((n, o), jnp.float32),
        compiler_params=_PAR,
    )(adj, feat_bf16, s, wn_bf16)


def _big_plain_body(a_ref, h_ref, o_ref):
    o_ref[...] = jnp.dot(a_ref[...].astype(jnp.bfloat16), h_ref[...],
                         preferred_element_type=jnp.float32)


def _spmm(adj, h_bf16, s=None, block_rows=400):
    n, k = adj.shape
    o = h_bf16.shape[1]
    r = block_rows if n % block_rows == 0 else n
    grid = (n // r,)
    a_spec = pl.BlockSpec((r, k), lambda i: (i, 0))
    h_spec = pl.BlockSpec((k, o), lambda i: (0, 0))
    out_spec = pl.BlockSpec((r, o), lambda i: (i, 0))
    out_shape = jax.ShapeDtypeStruct((n, o), jnp.float32)
    if s is None:
        return pl.pallas_call(
            _big_plain_body, grid=grid,
            in_specs=[a_spec, h_spec], out_specs=out_spec,
            out_shape=out_shape, compiler_params=_PAR,
        )(adj, h_bf16)
    s_spec = pl.BlockSpec((r, o), lambda i: (i, 0))
    return pl.pallas_call(
        _big_relu_body, grid=grid,
        in_specs=[a_spec, h_spec, s_spec], out_specs=out_spec,
        out_shape=out_shape, compiler_params=_PAR,
    )(adj, h_bf16, s)


def _mm_body(x_ref, w_ref, o_ref):
    o_ref[...] = jnp.dot(x_ref[...].astype(jnp.bfloat16),
                         w_ref[...].astype(jnp.bfloat16),
                         preferred_element_type=jnp.float32)


def _mm(x, w, block_rows=2000):
    n, d = x.shape
    o = w.shape[1]
    r = block_rows if n % block_rows == 0 else n
    return pl.pallas_call(
        _mm_body, grid=(n // r,),
        in_specs=[pl.BlockSpec((r, d), lambda i: (i, 0)),
                  pl.BlockSpec((d, o), lambda i: (0, 0))],
        out_specs=pl.BlockSpec((r, o), lambda i: (i, 0)),
        out_shape=jax.ShapeDtypeStruct((n, o), jnp.float32),
        compiler_params=_PAR,
    )(x, w)


def _attn_pair(e1, e2, w, u):
    bf = jnp.bfloat16
    f32 = jnp.float32
    dot = lambda a, b: jnp.dot(a.astype(bf), b.astype(bf),
                               preferred_element_type=f32)
    s1 = dot(jnp.tanh(dot(e1, w)), u)
    s2 = dot(jnp.tanh(dot(e2, w)), u)
    m = jnp.maximum(s1, s2)
    x1 = jnp.exp(s1 - m)
    x2 = jnp.exp(s2 - m)
    den = x1 + x2
    a1 = x1 / den
    a2 = x2 / den
    lat = (e1.astype(bf).astype(f32) * a1.astype(bf).astype(f32)
           + e2.astype(bf).astype(f32) * a2.astype(bf).astype(f32))
    return lat, a1, a2


def _attn_body(esp1_ref, eft1_ref, esp2_ref, eft2_ref,
               w1_ref, u1_ref, w2_ref, u2_ref, wc_ref, uc_ref,
               lat1_ref, lat2_ref, comb_ref, ap_ref):
    lat1, a10, a11 = _attn_pair(esp1_ref[...], eft1_ref[...],
                                w1_ref[...], u1_ref[...])
    lat2, a20, a21 = _attn_pair(esp2_ref[...], eft2_ref[...],
                                w2_ref[...], u2_ref[...])
    comb, ac0, ac1 = _attn_pair(lat1, lat2, wc_ref[...], uc_ref[...])
    lat1_ref[...] = lat1
    lat2_ref[...] = lat2
    comb_ref[...] = comb
    r = lat1.shape[0]
    ap_ref[...] = jnp.concatenate(
        [a10, a11, a20, a21, ac0, ac1,
         jnp.zeros((r, 122), jnp.float32)], axis=1)


def _attention(esp1, eft1, esp2, eft2, w1, u1, w2, u2, wc, uc,
               block_rows=2000):
    n, o = esp1.shape
    r = block_rows if n % block_rows == 0 else n
    row = pl.BlockSpec((r, o), lambda i: (i, 0))
    wspec = lambda a: pl.BlockSpec(a.shape, lambda i: (0, 0))
    f32 = jnp.float32
    return pl.pallas_call(
        _attn_body, grid=(n // r,),
        in_specs=[row, row, row, row,
                  wspec(w1), wspec(u1), wspec(w2), wspec(u2),
                  wspec(wc), wspec(uc)],
        out_specs=[row, row, row, pl.BlockSpec((r, 128), lambda i: (i, 0))],
        out_shape=[jax.ShapeDtypeStruct((n, o), f32),
                   jax.ShapeDtypeStruct((n, o), f32),
                   jax.ShapeDtypeStruct((n, o), f32),
                   jax.ShapeDtypeStruct((n, 128), f32)],
        compiler_params=_PAR,
    )(esp1, eft1, esp2, eft2, w1, u1, w2, u2, wc, uc)


def _epi_body(comb_ref, lat1_ref, lat2_ref, v1_ref, v2_ref,
              wsd1_ref, wnd1_ref, wsd2_ref, wnd2_ref,
              ws1_ref, wn1_ref, ws2_ref, wn2_ref,
              rec1_ref, rec2_ref, sa1_ref, ha1_ref, sa2_ref, ha2_ref):
    dot = lambda a, b: jnp.dot(a, b, preferred_element_type=jnp.float32)
    comb = comb_ref[...]
    o = comb.shape[1]
    v1 = v1_ref[...]
    v2 = v2_ref[...]
    rec1_ref[...] = jnp.maximum(
        dot(comb, wsd1_ref[...]) + dot(v1[:, :o], wnd1_ref[...]), 0.0)
    a2in = jnp.maximum(
        dot(lat2_ref[...], wsd1_ref[...]) + dot(v1[:, o:], wnd1_ref[...]), 0.0)
    rec2_ref[...] = jnp.maximum(
        dot(comb, wsd2_ref[...]) + dot(v2[:, :o], wnd2_ref[...]), 0.0)
    a1in = jnp.maximum(
        dot(lat1_ref[...], wsd2_ref[...]) + dot(v2[:, o:], wnd2_ref[...]), 0.0)
    sa1_ref[...] = dot(a1in, ws2_ref[...])
    ha1_ref[...] = dot(a1in, wn2_ref[...]).astype(jnp.bfloat16)
    sa2_ref[...] = dot(a2in, ws1_ref[...])
    ha2_ref[...] = dot(a2in, wn1_ref[...]).astype(jnp.bfloat16)


def _epilogue(comb, lat1, lat2, v1, v2,
              Wsd1, Wnd1, Wsd2, Wnd2, Ws1, Wn1, Ws2, Wn2,
              block_rows=2000):
    n, o = comb.shape
    d1 = Wsd1.shape[1]
    d2 = Wsd2.shape[1]
    r = block_rows if n % block_rows == 0 else n
    row = lambda c: pl.BlockSpec((r, c), lambda i: (i, 0))
    wspec = lambda a: pl.BlockSpec(a.shape, lambda i: (0, 0))
    f32 = jnp.float32
    return pl.pallas_call(
        _epi_body, grid=(n // r,),
        in_specs=[row(o), row(o), row(o), row(2 * o), row(2 * o),
                  wspec(Wsd1), wspec(Wnd1), wspec(Wsd2), wspec(Wnd2),
                  wspec(Ws1), wspec(Wn1), wspec(Ws2), wspec(Wn2)],
        out_specs=[row(d1), row(d2), row(o), row(o), row(o), row(o)],
        out_shape=[jax.ShapeDtypeStruct((n, d1), f32),
                   jax.ShapeDtypeStruct((n, d2), f32),
                   jax.ShapeDtypeStruct((n, o), f32),
                   jax.ShapeDtypeStruct((n, o), jnp.bfloat16),
                   jax.ShapeDtypeStruct((n, o), f32),
                   jax.ShapeDtypeStruct((n, o), jnp.bfloat16)],
        compiler_params=_PAR,
    )(comb, lat1, lat2, v1, v2,
      Wsd1, Wnd1, Wsd2, Wnd2, Ws1, Wn1, Ws2, Wn2)


def kernel(features_omics1, features_omics2, adj_spatial_omics1,
           adj_feature_omics1, adj_spatial_omics2, adj_feature_omics2,
           Ws1, Wn1, Wsd1, Wnd1, Ws2, Wn2, Wsd2, Wnd2,
           w1, u1, w2, u2, wc, uc):
    s1 = _mm(features_omics1, Ws1)
    s2 = _mm(features_omics2, Ws2)
    f1b = features_omics1.astype(jnp.bfloat16)
    f2b = features_omics2.astype(jnp.bfloat16)
    wn1b = Wn1.astype(jnp.bfloat16)
    wn2b = Wn2.astype(jnp.bfloat16)

    e_sp1 = _sage_ref(adj_spatial_omics1, f1b, s1, wn1b)
    e_ft1 = _sage_ref(adj_feature_omics1, f1b, s1, wn1b)
    e_sp2 = _sage_ref(adj_spatial_omics2, f2b, s2, wn2b)
    e_ft2 = _sage_ref(adj_feature_omics2, f2b, s2, wn2b)

    lat1, lat2, comb, ap = _attention(e_sp1, e_ft1, e_sp2, e_ft2,
                                      w1, u1, w2, u2, wc, uc)
    alpha1 = ap[:, 0:2]
    alpha2 = ap[:, 2:4]
    alpha12 = ap[:, 4:6]

    cu1 = jnp.concatenate([comb, lat2], axis=1).astype(jnp.bfloat16)
    cu2 = jnp.concatenate([comb, lat1], axis=1).astype(jnp.bfloat16)
    v1 = _spmm(adj_spatial_omics1, cu1)
    v2 = _spmm(adj_spatial_omics2, cu2)

    rec1, rec2, sa1, ha1, sa2, ha2 = _epilogue(
        comb, lat1, lat2, v1, v2,
        Wsd1, Wnd1, Wsd2, Wnd2, Ws1, Wn1, Ws2, Wn2)

    across1 = _spmm(adj_spatial_omics2, ha1, sa1)
    across2 = _spmm(adj_spatial_omics1, ha2, sa2)

    return (lat1, lat2, comb, rec1, rec2, across1, across2,
            alpha1, alpha2, alpha12)
